# baseline (speedup 1.0000x reference)
import functools

import jax
import jax.numpy as jnp
from jax import lax
from jax.experimental import pallas as pl
from jax.experimental.pallas import tpu as pltpu
from jax.experimental.pallas import tpu_sc as plsc

N = 10000
NP = 10240
D = 128
E = 320000
NC, NS = 2, 16
NW = NC * NS
CHUNK = 128
CPT = 80
E_PAD = NW * CPT * CHUNK
DUMP = N + 100
RPT = NP // NS

_vmesh = plsc.VectorSubcoreMesh(core_axis_name="c", subcore_axis_name="s")


@functools.partial(
    pl.kernel,
    out_type=jax.ShapeDtypeStruct((NC, NP, D), jnp.float32),
    mesh=_vmesh,
    scratch_types=[
        pltpu.VMEM((CPT, CHUNK), jnp.int32),
        pltpu.VMEM((CHUNK, D), jnp.float32),
        pltpu.VMEM_SHARED((NP, D), jnp.float32),
        pltpu.SemaphoreType.DMA,
    ],
)
def _deg_kernel(row_hbm, zeros_hbm, out_hbm, idx_v, ones_v, acc_sh, sem):
    cid = lax.axis_index("c")
    sid = lax.axis_index("s")
    wid = cid * NS + sid

    pltpu.async_copy(row_hbm.at[wid], idx_v, sem).wait()

    @pl.loop(0, CHUNK)
    def _(i):
        @pl.loop(0, D, step=16)
        def _(f):
            ones_v[i, pl.ds(f, 16)] = jnp.full((16,), 1.0, jnp.float32)

    pltpu.sync_copy(zeros_hbm.at[pl.ds(sid * RPT, RPT)],
                    acc_sh.at[pl.ds(sid * RPT, RPT)])
    plsc.subcore_barrier()

    @pl.loop(0, CPT)
    def _(j):
        pltpu.async_copy(ones_v, acc_sh.at[idx_v.at[j]], sem, add=True)

    @pl.loop(0, CPT)
    def _(j):
        pltpu.make_async_copy(ones_v, acc_sh.at[idx_v.at[j]], sem).wait()

    plsc.subcore_barrier()
    pltpu.sync_copy(acc_sh.at[pl.ds(sid * RPT, RPT)],
                    out_hbm.at[cid, pl.ds(sid * RPT, RPT)])


CPT2 = CPT // 2


@functools.partial(
    pl.kernel,
    out_type=jax.ShapeDtypeStruct((NC, NP, D), jnp.float32),
    mesh=_vmesh,
    scratch_types=[
        pltpu.VMEM((CPT2, CHUNK), jnp.int32),
        pltpu.VMEM((CPT2, CHUNK), jnp.int32),
        pltpu.VMEM((CHUNK, D), jnp.float32),
        pltpu.VMEM((CHUNK, D), jnp.float32),
        pltpu.VMEM_SHARED((NP, D), jnp.float32),
        pltpu.SemaphoreType.DMA,
        pltpu.SemaphoreType.DMA,
        pltpu.SemaphoreType.DMA,
        pltpu.SemaphoreType.DMA,
        pltpu.SemaphoreType.DMA,
    ],
)
def _agg_kernel(g_hbm, row_hbm, col_hbm, zeros_hbm, out_hbm,
                row_v, col_v, buf0, buf1, acc_sh, sg0, sg1, ss0, ss1, si):
    cid = lax.axis_index("c")
    sid = lax.axis_index("s")
    wid = cid * NS + sid

    pltpu.sync_copy(zeros_hbm.at[pl.ds(sid * RPT, RPT)],
                    acc_sh.at[pl.ds(sid * RPT, RPT)])
    plsc.subcore_barrier()

    for phase in range(CPT // CPT2):
        base = phase * CPT2
        pltpu.async_copy(row_hbm.at[wid, pl.ds(base, CPT2)], row_v, si).wait()
        pltpu.async_copy(col_hbm.at[wid, pl.ds(base, CPT2)], col_v, si).wait()

        pltpu.async_copy(g_hbm.at[row_v.at[0]], buf0, sg0)
        pltpu.async_copy(g_hbm.at[row_v.at[1]], buf1, sg1)

        @pl.loop(0, CPT2 - 2, step=2)
        def _(j):
            pltpu.make_async_copy(g_hbm.at[row_v.at[j]], buf0, sg0).wait()
            pltpu.async_copy(buf0, acc_sh.at[col_v.at[j]], ss0, add=True)
            pltpu.make_async_copy(g_hbm.at[row_v.at[j + 1]], buf1, sg1).wait()
            pltpu.async_copy(buf1, acc_sh.at[col_v.at[j + 1]], ss1, add=True)
            pltpu.make_async_copy(buf0, acc_sh.at[col_v.at[j]], ss0).wait()
            pltpu.async_copy(g_hbm.at[row_v.at[j + 2]], buf0, sg0)
            pltpu.make_async_copy(buf1, acc_sh.at[col_v.at[j + 1]], ss1).wait()
            pltpu.async_copy(g_hbm.at[row_v.at[j + 3]], buf1, sg1)

        pltpu.make_async_copy(g_hbm.at[row_v.at[CPT2 - 2]], buf0, sg0).wait()
        pltpu.async_copy(buf0, acc_sh.at[col_v.at[CPT2 - 2]], ss0, add=True)
        pltpu.make_async_copy(g_hbm.at[row_v.at[CPT2 - 1]], buf1, sg1).wait()
        pltpu.async_copy(buf1, acc_sh.at[col_v.at[CPT2 - 1]], ss1, add=True)
        pltpu.make_async_copy(buf0, acc_sh.at[col_v.at[CPT2 - 2]], ss0).wait()
        pltpu.make_async_copy(buf1, acc_sh.at[col_v.at[CPT2 - 1]], ss1).wait()

    plsc.subcore_barrier()
    pltpu.sync_copy(acc_sh.at[pl.ds(sid * RPT, RPT)],
                    out_hbm.at[cid, pl.ds(sid * RPT, RPT)])


_R = 512


def _prep_body(deg_ref, x_ref, w_ref, b_ref, g_ref, d_ref):
    deg = deg_ref[0, :, 0:1] + deg_ref[1, :, 0:1]
    d = jnp.where(deg > 0, lax.rsqrt(deg), 0.0)
    h = lax.dot_general(x_ref[...], w_ref[...], (((1,), (1,)), ((), ())),
                        preferred_element_type=jnp.float32) + b_ref[...]
    d_ref[...] = d
    g_ref[...] = d * h


def _prep(deg_parts, x_pad, w, b):
    return pl.pallas_call(
        _prep_body,
        grid=(NP // _R,),
        in_specs=[
            pl.BlockSpec((NC, _R, D), lambda i: (0, i, 0)),
            pl.BlockSpec((_R, D), lambda i: (i, 0)),
            pl.BlockSpec((D, D), lambda i: (0, 0)),
            pl.BlockSpec((1, D), lambda i: (0, 0)),
        ],
        out_specs=[
            pl.BlockSpec((_R, D), lambda i: (i, 0)),
            pl.BlockSpec((_R, 1), lambda i: (i, 0)),
        ],
        out_shape=[
            jax.ShapeDtypeStruct((NP, D), jnp.float32),
            jax.ShapeDtypeStruct((NP, 1), jnp.float32),
        ],
    )(deg_parts, x_pad, w, b)


def _mid_body(p_ref, d_ref, w_ref, b_ref, g_ref):
    d = d_ref[...]
    y = jnp.maximum(d * (p_ref[0] + p_ref[1]), 0.0)
    h = lax.dot_general(y, w_ref[...], (((1,), (1,)), ((), ())),
                        preferred_element_type=jnp.float32) + b_ref[...]
    g_ref[...] = d * h


def _mid(parts, d, w, b):
    return pl.pallas_call(
        _mid_body,
        grid=(NP // _R,),
        in_specs=[
            pl.BlockSpec((NC, _R, D), lambda i: (0, i, 0)),
            pl.BlockSpec((_R, 1), lambda i: (i, 0)),
            pl.BlockSpec((D, D), lambda i: (0, 0)),
            pl.BlockSpec((1, D), lambda i: (0, 0)),
        ],
        out_specs=pl.BlockSpec((_R, D), lambda i: (i, 0)),
        out_shape=jax.ShapeDtypeStruct((NP, D), jnp.float32),
    )(parts, d, w, b)


def _fin_body(p_ref, d_ref, o_ref):
    o_ref[...] = jnp.maximum(d_ref[...] * (p_ref[0] + p_ref[1]), 0.0)


def _fin(parts, d):
    return pl.pallas_call(
        _fin_body,
        grid=(NP // _R,),
        in_specs=[
            pl.BlockSpec((NC, _R, D), lambda i: (0, i, 0)),
            pl.BlockSpec((_R, 1), lambda i: (i, 0)),
        ],
        out_specs=pl.BlockSpec((_R, D), lambda i: (i, 0)),
        out_shape=jax.ShapeDtypeStruct((NP, D), jnp.float32),
    )(parts, d)


def kernel(x, edge_index, edge_attr, W1, b1, W2, b2, W3, b3):
    row = edge_index[0].astype(jnp.int32)
    col = edge_index[1].astype(jnp.int32)
    pad = N + jnp.arange(E_PAD - E, dtype=jnp.int32) % (NP - N)
    row_p = jnp.concatenate([row, pad]).reshape(NW, CPT, CHUNK)
    col_p = jnp.concatenate([col, pad]).reshape(NW, CPT, CHUNK)
    x_pad = jnp.pad(x, ((0, NP - N), (0, 0)))
    zeros = jnp.zeros((NP, D), jnp.float32)

    deg_parts = _deg_kernel(row_p, zeros)
    g, d = _prep(deg_parts, x_pad, W1, b1.reshape(1, D))
    parts = _agg_kernel(g, row_p, col_p, zeros)
    g = _mid(parts, d, W2, b2.reshape(1, D))
    parts = _agg_kernel(g, row_p, col_p, zeros)
    g = _mid(parts, d, W3, b3.reshape(1, D))
    parts = _agg_kernel(g, row_p, col_p, zeros)
    out = _fin(parts, d)
    return out[:N]


# speedup vs baseline: 1.2196x; 1.2196x over previous
import functools

import jax
import jax.numpy as jnp
from jax import lax
from jax.experimental import pallas as pl
from jax.experimental.pallas import tpu as pltpu
from jax.experimental.pallas import tpu_sc as plsc

N = 10000
NP = 10240
D = 128
E = 320000
NC, NS = 2, 16
NW = NC * NS
CHUNK = 128
CPT = 80
E_PAD = NW * CPT * CHUNK
DUMP = N + 100
RPT = NP // NS

_vmesh = plsc.VectorSubcoreMesh(core_axis_name="c", subcore_axis_name="s")


@functools.partial(
    pl.kernel,
    out_type=jax.ShapeDtypeStruct((NC, NP, D), jnp.float32),
    mesh=_vmesh,
    scratch_types=[
        pltpu.VMEM((CPT, CHUNK), jnp.int32),
        pltpu.VMEM((CHUNK, D), jnp.float32),
        pltpu.VMEM_SHARED((NP, D), jnp.float32),
        pltpu.SemaphoreType.DMA,
    ],
)
def _deg_kernel(row_hbm, zeros_hbm, out_hbm, idx_v, ones_v, acc_sh, sem):
    cid = lax.axis_index("c")
    sid = lax.axis_index("s")
    wid = cid * NS + sid

    pltpu.async_copy(row_hbm.at[wid], idx_v, sem).wait()

    @pl.loop(0, CHUNK)
    def _(i):
        @pl.loop(0, D, step=16)
        def _(f):
            ones_v[i, pl.ds(f, 16)] = jnp.full((16,), 1.0, jnp.float32)

    pltpu.sync_copy(zeros_hbm.at[pl.ds(sid * RPT, RPT)],
                    acc_sh.at[pl.ds(sid * RPT, RPT)])
    plsc.subcore_barrier()

    @pl.loop(0, CPT)
    def _(j):
        pltpu.async_copy(ones_v, acc_sh.at[idx_v.at[j]], sem, add=True)

    @pl.loop(0, CPT)
    def _(j):
        pltpu.make_async_copy(ones_v, acc_sh.at[idx_v.at[j]], sem).wait()

    plsc.subcore_barrier()
    pltpu.sync_copy(acc_sh.at[pl.ds(sid * RPT, RPT)],
                    out_hbm.at[cid, pl.ds(sid * RPT, RPT)])


CPT2 = CPT // 2


@functools.partial(
    pl.kernel,
    out_type=jax.ShapeDtypeStruct((NC, NP, D), jnp.float32),
    mesh=_vmesh,
    scratch_types=[
        pltpu.VMEM((CPT2, CHUNK), jnp.int32),
        pltpu.VMEM((CPT2, CHUNK), jnp.int32),
        pltpu.VMEM((CHUNK, D), jnp.float32),
        pltpu.VMEM((CHUNK, D), jnp.float32),
        pltpu.VMEM_SHARED((NP, D), jnp.float32),
        pltpu.SemaphoreType.DMA,
        pltpu.SemaphoreType.DMA,
        pltpu.SemaphoreType.DMA,
        pltpu.SemaphoreType.DMA,
        pltpu.SemaphoreType.DMA,
    ],
)
def _agg_kernel(g_hbm, row_hbm, col_hbm, zeros_hbm, out_hbm,
                row_v, col_v, buf0, buf1, acc_sh, sg0, sg1, ss0, ss1, si):
    cid = lax.axis_index("c")
    sid = lax.axis_index("s")
    wid = cid * NS + sid

    pltpu.sync_copy(zeros_hbm.at[pl.ds(sid * RPT, RPT)],
                    acc_sh.at[pl.ds(sid * RPT, RPT)])
    plsc.subcore_barrier()

    for phase in range(CPT // CPT2):
        base = phase * CPT2
        pltpu.async_copy(row_hbm.at[wid, pl.ds(base, CPT2)], row_v, si).wait()
        pltpu.async_copy(col_hbm.at[wid, pl.ds(base, CPT2)], col_v, si).wait()

        pltpu.async_copy(g_hbm.at[row_v.at[0]], buf0, sg0)
        pltpu.async_copy(g_hbm.at[row_v.at[1]], buf1, sg1)

        @pl.loop(0, CPT2 - 2, step=2)
        def _(j):
            pltpu.make_async_copy(g_hbm.at[row_v.at[j]], buf0, sg0).wait()
            pltpu.sync_copy(buf0, acc_sh.at[col_v.at[j]], add=True)
            pltpu.async_copy(g_hbm.at[row_v.at[j + 2]], buf0, sg0)
            pltpu.make_async_copy(g_hbm.at[row_v.at[j + 1]], buf1, sg1).wait()
            pltpu.sync_copy(buf1, acc_sh.at[col_v.at[j + 1]], add=True)
            pltpu.async_copy(g_hbm.at[row_v.at[j + 3]], buf1, sg1)

        pltpu.make_async_copy(g_hbm.at[row_v.at[CPT2 - 2]], buf0, sg0).wait()
        pltpu.sync_copy(buf0, acc_sh.at[col_v.at[CPT2 - 2]], add=True)
        pltpu.make_async_copy(g_hbm.at[row_v.at[CPT2 - 1]], buf1, sg1).wait()
        pltpu.sync_copy(buf1, acc_sh.at[col_v.at[CPT2 - 1]], add=True)

    plsc.subcore_barrier()
    pltpu.sync_copy(acc_sh.at[pl.ds(sid * RPT, RPT)],
                    out_hbm.at[cid, pl.ds(sid * RPT, RPT)])


_R = 512


def _prep_body(deg_ref, x_ref, w_ref, b_ref, g_ref, d_ref):
    deg = deg_ref[0, :, 0:1] + deg_ref[1, :, 0:1]
    d = jnp.where(deg > 0, lax.rsqrt(deg), 0.0)
    h = lax.dot_general(x_ref[...], w_ref[...], (((1,), (1,)), ((), ())),
                        preferred_element_type=jnp.float32) + b_ref[...]
    d_ref[...] = d
    g_ref[...] = d * h


def _prep(deg_parts, x_pad, w, b):
    return pl.pallas_call(
        _prep_body,
        grid=(NP // _R,),
        in_specs=[
            pl.BlockSpec((NC, _R, D), lambda i: (0, i, 0)),
            pl.BlockSpec((_R, D), lambda i: (i, 0)),
            pl.BlockSpec((D, D), lambda i: (0, 0)),
            pl.BlockSpec((1, D), lambda i: (0, 0)),
        ],
        out_specs=[
            pl.BlockSpec((_R, D), lambda i: (i, 0)),
            pl.BlockSpec((_R, 1), lambda i: (i, 0)),
        ],
        out_shape=[
            jax.ShapeDtypeStruct((NP, D), jnp.float32),
            jax.ShapeDtypeStruct((NP, 1), jnp.float32),
        ],
    )(deg_parts, x_pad, w, b)


def _mid_body(p_ref, d_ref, w_ref, b_ref, g_ref):
    d = d_ref[...]
    y = jnp.maximum(d * (p_ref[0] + p_ref[1]), 0.0)
    h = lax.dot_general(y, w_ref[...], (((1,), (1,)), ((), ())),
                        preferred_element_type=jnp.float32) + b_ref[...]
    g_ref[...] = d * h


def _mid(parts, d, w, b):
    return pl.pallas_call(
        _mid_body,
        grid=(NP // _R,),
        in_specs=[
            pl.BlockSpec((NC, _R, D), lambda i: (0, i, 0)),
            pl.BlockSpec((_R, 1), lambda i: (i, 0)),
            pl.BlockSpec((D, D), lambda i: (0, 0)),
            pl.BlockSpec((1, D), lambda i: (0, 0)),
        ],
        out_specs=pl.BlockSpec((_R, D), lambda i: (i, 0)),
        out_shape=jax.ShapeDtypeStruct((NP, D), jnp.float32),
    )(parts, d, w, b)


def _fin_body(p_ref, d_ref, o_ref):
    o_ref[...] = jnp.maximum(d_ref[...] * (p_ref[0] + p_ref[1]), 0.0)


def _fin(parts, d):
    return pl.pallas_call(
        _fin_body,
        grid=(NP // _R,),
        in_specs=[
            pl.BlockSpec((NC, _R, D), lambda i: (0, i, 0)),
            pl.BlockSpec((_R, 1), lambda i: (i, 0)),
        ],
        out_specs=pl.BlockSpec((_R, D), lambda i: (i, 0)),
        out_shape=jax.ShapeDtypeStruct((NP, D), jnp.float32),
    )(parts, d)


def kernel(x, edge_index, edge_attr, W1, b1, W2, b2, W3, b3):
    row = edge_index[0].astype(jnp.int32)
    col = edge_index[1].astype(jnp.int32)
    pad = N + jnp.arange(E_PAD - E, dtype=jnp.int32) % (NP - N)
    row_p = jnp.concatenate([row, pad]).reshape(NW, CPT, CHUNK)
    col_p = jnp.concatenate([col, pad]).reshape(NW, CPT, CHUNK)
    x_pad = jnp.pad(x, ((0, NP - N), (0, 0)))
    zeros = jnp.zeros((NP, D), jnp.float32)

    deg_parts = _deg_kernel(row_p, zeros)
    g, d = _prep(deg_parts, x_pad, W1, b1.reshape(1, D))
    parts = _agg_kernel(g, row_p, col_p, zeros)
    g = _mid(parts, d, W2, b2.reshape(1, D))
    parts = _agg_kernel(g, row_p, col_p, zeros)
    g = _mid(parts, d, W3, b3.reshape(1, D))
    parts = _agg_kernel(g, row_p, col_p, zeros)
    out = _fin(parts, d)
    return out[:N]
